# baseline (speedup 1.0000x reference)
import functools

import jax
import jax.numpy as jnp
from jax.experimental import pallas as pl
from jax.experimental.pallas import tpu as pltpu


def _matmul_body(x_ref, w_ref, s_ref, b_ref, o_ref, *, n_k):
    k = pl.program_id(2)
    w_bf = (w_ref[...].astype(jnp.float32) * s_ref[0]).astype(jnp.bfloat16)
    part = jax.lax.dot_general(
        x_ref[...], w_bf,
        dimension_numbers=(((1,), (0,)), ((), ())),
        preferred_element_type=jnp.float32,
    )

    @pl.when(k == 0)
    def _init():
        o_ref[...] = part + b_ref[...]

    @pl.when(k > 0)
    def _acc():
        o_ref[...] += part


def _quant_matmul(x_bf, w_t, s_t, b_row, *, bm, bn, bk):
    m, kdim = x_bf.shape
    n = w_t.shape[1]
    n_k = kdim // bk
    grid = (pl.cdiv(n, bn), pl.cdiv(m, bm), n_k)
    return pl.pallas_call(
        functools.partial(_matmul_body, n_k=n_k),
        grid=grid,
        in_specs=[
            pl.BlockSpec((bm, bk), lambda ni, mi, ki: (mi, ki)),
            pl.BlockSpec((bk, bn), lambda ni, mi, ki: (ki, ni)),
            pl.BlockSpec((1, 1, bn), lambda ni, mi, ki: (ki, 0, ni)),
            pl.BlockSpec((1, bn), lambda ni, mi, ki: (0, ni)),
        ],
        out_specs=pl.BlockSpec((bm, bn), lambda ni, mi, ki: (mi, ni)),
        out_shape=jax.ShapeDtypeStruct((m, n), jnp.float32),
        compiler_params=pltpu.CompilerParams(
            dimension_semantics=("parallel", "parallel", "arbitrary"),
        ),
    )(x_bf, w_t, s_t, b_row)


def kernel(x, w_q, scales, bias):
    out_f, n_groups, group = w_q.shape
    m, in_f = x.shape
    w_t = w_q.reshape(out_f, in_f).T
    s_t = scales.reshape(out_f, n_groups).T.reshape(n_groups, 1, out_f)
    b_row = bias.reshape(1, out_f)
    x_bf = x.astype(jnp.bfloat16)
    y = _quant_matmul(x_bf, w_t, s_t, b_row, bm=1024, bn=512, bk=group)
    return y.astype(x.dtype)


# speedup vs baseline: 5.1713x; 5.1713x over previous
import jax
import jax.numpy as jnp
from jax.experimental import pallas as pl
from jax.experimental.pallas import tpu as pltpu


def _matmul_body(x_ref, w_ref, s_ref, b_ref, o_ref):
    n_groups, group, bn = w_ref.shape
    w_bf = (w_ref[...].astype(jnp.float32) * s_ref[...]).astype(jnp.bfloat16)
    w_bf = w_bf.reshape(n_groups * group, bn)
    o_ref[...] = jax.lax.dot_general(
        x_ref[...], w_bf,
        dimension_numbers=(((1,), (0,)), ((), ())),
        preferred_element_type=jnp.float32,
    ) + b_ref[...]


def _quant_matmul(x_bf, w_t, s_t, b_row, *, bm, bn):
    m, kdim = x_bf.shape
    n_groups, group, n = w_t.shape
    grid = (pl.cdiv(m, bm), pl.cdiv(n, bn))
    return pl.pallas_call(
        _matmul_body,
        grid=grid,
        in_specs=[
            pl.BlockSpec((bm, kdim), lambda mi, ni: (mi, 0)),
            pl.BlockSpec((n_groups, group, bn), lambda mi, ni: (0, 0, ni)),
            pl.BlockSpec((n_groups, 1, bn), lambda mi, ni: (0, 0, ni)),
            pl.BlockSpec((1, bn), lambda mi, ni: (0, ni)),
        ],
        out_specs=pl.BlockSpec((bm, bn), lambda mi, ni: (mi, ni)),
        out_shape=jax.ShapeDtypeStruct((m, n), jnp.float32),
        compiler_params=pltpu.CompilerParams(
            dimension_semantics=("parallel", "parallel"),
        ),
    )(x_bf, w_t, s_t, b_row)


def kernel(x, w_q, scales, bias):
    out_f, n_groups, group = w_q.shape
    m, in_f = x.shape
    w_t = jnp.transpose(w_q, (1, 2, 0))
    s_t = scales.reshape(out_f, n_groups).T.reshape(n_groups, 1, out_f)
    b_row = bias.reshape(1, out_f)
    x_bf = x.astype(jnp.bfloat16)
    y = _quant_matmul(x_bf, w_t, s_t, b_row, bm=1024, bn=512)
    return y.astype(x.dtype)


# speedup vs baseline: 5.2951x; 1.0239x over previous
import jax
import jax.numpy as jnp
from jax.experimental import pallas as pl
from jax.experimental.pallas import tpu as pltpu


def _matmul_body(x_ref, w_ref, s_ref, b_ref, o_ref):
    n_groups, group, bn = w_ref.shape
    w_bf = (w_ref[...].astype(jnp.float32) * s_ref[...]).astype(jnp.bfloat16)
    w_bf = w_bf.reshape(n_groups * group, bn)
    o_ref[...] = jax.lax.dot_general(
        x_ref[...], w_bf,
        dimension_numbers=(((1,), (0,)), ((), ())),
        preferred_element_type=jnp.float32,
    ) + b_ref[...]


def _quant_matmul(x_bf, w_t, s_t, b_row, *, bm, bn):
    m, kdim = x_bf.shape
    n_groups, group, n = w_t.shape
    grid = (pl.cdiv(m, bm), pl.cdiv(n, bn))
    return pl.pallas_call(
        _matmul_body,
        grid=grid,
        in_specs=[
            pl.BlockSpec((bm, kdim), lambda mi, ni: (mi, 0)),
            pl.BlockSpec((n_groups, group, bn), lambda mi, ni: (0, 0, ni)),
            pl.BlockSpec((n_groups, 1, bn), lambda mi, ni: (0, 0, ni)),
            pl.BlockSpec((1, bn), lambda mi, ni: (0, ni)),
        ],
        out_specs=pl.BlockSpec((bm, bn), lambda mi, ni: (mi, ni)),
        out_shape=jax.ShapeDtypeStruct((m, n), jnp.float32),
        compiler_params=pltpu.CompilerParams(
            dimension_semantics=("parallel", "parallel"),
        ),
    )(x_bf, w_t, s_t, b_row)


def kernel(x, w_q, scales, bias):
    out_f, n_groups, group = w_q.shape
    m, in_f = x.shape
    w_t = jnp.transpose(w_q, (1, 2, 0))
    s_t = scales.reshape(out_f, n_groups).T.reshape(n_groups, 1, out_f)
    b_row = bias.reshape(1, out_f)
    x_bf = x.astype(jnp.bfloat16)
    y = _quant_matmul(x_bf, w_t, s_t, b_row, bm=2048, bn=512)
    return y.astype(x.dtype)
